# baseline (speedup 1.0000x reference)
import functools

import jax
import jax.numpy as jnp
from jax import lax
from jax.experimental import pallas as pl
from jax.experimental.pallas import tpu as pltpu
from jax.experimental.pallas import tpu_sc as plsc

N_NODES = 10000
N_EDGES = 320000
D_FEAT = 128
MAX_HOP = 3

NC = 2
NS = 16
N_PAD = 10240
NODES_PER_CORE = N_PAD // NC
ROWS_PER_TILE = NODES_PER_CORE // NS
DUMP_ROW = NODES_PER_CORE
ACC_ROWS = NODES_PER_CORE + 8
CHUNK = 80
BLOCK = NS * CHUNK
E_PAD = N_EDGES + 2 * BLOCK
N_CHUNKS = E_PAD // CHUNK


def _sc_body(table, packed, trips, zeros, out, acc, rows, pk, sem0, sem1):
    c = lax.axis_index("c")
    s = lax.axis_index("s")
    sems = (sem0, sem1)

    pltpu.sync_copy(trips.at[c], pk.at[0, 0])
    half = pk[0, 0, pl.ds(0, 16)][0]

    row0 = s * ROWS_PER_TILE
    out_row0 = c * NODES_PER_CORE + s * ROWS_PER_TILE

    pltpu.sync_copy(zeros, acc.at[pl.ds(row0, ROWS_PER_TILE)])
    plsc.subcore_barrier()

    def idx_src(j):
        return packed.at[c, s + NS * j]

    def issue_idx(j, b):
        pltpu.async_copy(idx_src(j), pk.at[b], sems[b])

    def drain_idx(b):
        pltpu.make_async_copy(packed.at[0, 0], pk.at[b], sems[b]).wait()

    def issue_gather(b):
        pltpu.async_copy(table.at[pk.at[b, 0]], rows.at[b], sems[b])

    def drain_rows(b):
        pltpu.make_async_copy(zeros.at[pl.ds(0, CHUNK)], rows.at[b],
                              sems[b]).wait()

    issue_idx(0, 0)
    drain_idx(0)
    issue_gather(0)
    issue_idx(1, 1)

    def pipe_body(i, _):
        for b in range(2):
            j = 2 * i + b
            o = 1 - b
            drain_idx(o)
            issue_gather(o)
            drain_rows(b)
            pltpu.sync_copy(rows.at[b], acc.at[pk.at[b, 1]], add=True)
            issue_idx(j + 2, b)
        return ()

    lax.fori_loop(0, half, pipe_body, ())
    drain_idx(1)
    drain_rows(0)

    plsc.subcore_barrier()
    pltpu.sync_copy(acc.at[pl.ds(row0, ROWS_PER_TILE)],
                    out.at[pl.ds(out_row0, ROWS_PER_TILE)])


@functools.partial(
    pl.kernel,
    out_type=jax.ShapeDtypeStruct((N_PAD, D_FEAT), jnp.float32),
    mesh=plsc.VectorSubcoreMesh(core_axis_name="c", subcore_axis_name="s"),
    scratch_types=[
        pltpu.VMEM_SHARED((ACC_ROWS, D_FEAT), jnp.float32),
        pltpu.VMEM((2, CHUNK, D_FEAT), jnp.float32),
        pltpu.VMEM((2, 2, CHUNK), jnp.int32),
        pltpu.SemaphoreType.DMA,
        pltpu.SemaphoreType.DMA,
    ],
)
def _hop_kernel(table, packed, trips, zeros, out, acc, rows, pk, sem0, sem1):
    _sc_body(table, packed, trips, zeros, out, acc, rows, pk, sem0, sem1)


def kernel(x, edge_index, num_nodes):
    del num_nodes
    src = edge_index[0]
    dst = edge_index[1]
    core = src // NODES_PER_CORE
    local = src - core * NODES_PER_CORE
    ones1 = jnp.cumsum(core)
    pos = jnp.where(core == 0, jnp.arange(N_EDGES) - ones1, ones1 - 1)
    flat = core * E_PAD + pos
    src_arr = jnp.full((NC * E_PAD,), DUMP_ROW, jnp.int32).at[flat].set(local)
    dst_arr = jnp.zeros((NC * E_PAD,), jnp.int32).at[flat].set(dst)
    packed = jnp.stack(
        [dst_arr.reshape(NC, N_CHUNKS, CHUNK),
         src_arr.reshape(NC, N_CHUNKS, CHUNK)], axis=2)
    n1 = ones1[-1]
    counts = jnp.stack([N_EDGES - n1, n1])
    half = jnp.maximum((counts + 2 * BLOCK - 1) // (2 * BLOCK), 1)
    trips = jnp.broadcast_to(half.astype(jnp.int32)[:, None], (NC, CHUNK))
    zeros = jnp.zeros((ROWS_PER_TILE, D_FEAT), jnp.float32)

    table = jnp.pad(x, ((0, N_PAD - N_NODES), (0, 0)))
    hops = []
    for _ in range(MAX_HOP):
        table = _hop_kernel(table, packed, trips, zeros)
        hops.append(table[:N_NODES])
    y = jnp.stack(hops)
    return jnp.concatenate([x[:, None], jnp.transpose(y, (1, 0, 2))], axis=1)


# speedup vs baseline: 2.1864x; 2.1864x over previous
import functools

import jax
import jax.numpy as jnp
from jax import lax
from jax.experimental import pallas as pl
from jax.experimental.pallas import tpu as pltpu
from jax.experimental.pallas import tpu_sc as plsc

N_NODES = 10000
N_EDGES = 320000
D_FEAT = 128
MAX_HOP = 3

NC = 2
NS = 16
N_PAD = 10240
ROWS_PER_TILE = N_PAD // NS
DUMP_ROW = N_PAD
ACC_ROWS = N_PAD + 8
EDGES_PER_TILE = N_EDGES // (NC * NS)
CHUNK = 80
CHUNKS_REAL = EDGES_PER_TILE // CHUNK
PHASE = 16
N_PHASES = 8
CHUNKS_CAP = PHASE * N_PHASES


def _sc_body(table, packed, zeros, out, acc, rows, pkv):
    c = lax.axis_index("c")
    s = lax.axis_index("s")

    row0 = s * ROWS_PER_TILE
    pltpu.sync_copy(zeros, acc.at[pl.ds(row0, ROWS_PER_TILE)])
    plsc.subcore_barrier()

    my_idx = packed.at[c, s]

    def phase_body(p, _):
        pltpu.sync_copy(my_idx.at[pl.ds(p * PHASE, PHASE)], pkv)

        def chunk_body(j, _):
            pltpu.sync_copy(table.at[pkv.at[j, 0]], rows)
            pltpu.sync_copy(rows, acc.at[pkv.at[j, 1]], add=True)
            return ()

        lax.fori_loop(0, PHASE, chunk_body, ())
        return ()

    lax.fori_loop(0, N_PHASES, phase_body, ())

    plsc.subcore_barrier()
    pltpu.sync_copy(acc.at[pl.ds(row0, ROWS_PER_TILE)],
                    out.at[c, pl.ds(row0, ROWS_PER_TILE)])


@functools.partial(
    pl.kernel,
    out_type=jax.ShapeDtypeStruct((NC, N_PAD, D_FEAT), jnp.float32),
    mesh=plsc.VectorSubcoreMesh(core_axis_name="c", subcore_axis_name="s"),
    scratch_types=[
        pltpu.VMEM_SHARED((ACC_ROWS, D_FEAT), jnp.float32),
        pltpu.VMEM((CHUNK, D_FEAT), jnp.float32),
        pltpu.VMEM((PHASE, 2, CHUNK), jnp.int32),
    ],
)
def _partial_kernel(table, packed, zeros, out, acc, rows, pkv):
    _sc_body(table, packed, zeros, out, acc, rows, pkv)


def _add_body(p_ref, o_ref):
    o_ref[...] = p_ref[0] + p_ref[1]


_combine = pl.pallas_call(
    _add_body,
    out_shape=jax.ShapeDtypeStruct((N_PAD, D_FEAT), jnp.float32),
)


def kernel(x, edge_index, num_nodes):
    del num_nodes
    src = edge_index[0]
    dst = edge_index[1]
    dst4 = dst.reshape(NC, NS, CHUNKS_REAL, CHUNK)
    src4 = src.reshape(NC, NS, CHUNKS_REAL, CHUNK)
    pad = ((0, 0), (0, 0), (0, CHUNKS_CAP - CHUNKS_REAL), (0, 0))
    packed = jnp.stack(
        [jnp.pad(dst4, pad),
         jnp.pad(src4, pad, constant_values=DUMP_ROW)], axis=3)
    zeros = jnp.zeros((ROWS_PER_TILE, D_FEAT), jnp.float32)

    table = jnp.pad(x, ((0, N_PAD - N_NODES), (0, 0)))
    hops = []
    for _ in range(MAX_HOP):
        partials = _partial_kernel(table, packed, zeros)
        table = _combine(partials)
        hops.append(table[:N_NODES])
    y = jnp.stack(hops)
    return jnp.concatenate([x[:, None], jnp.transpose(y, (1, 0, 2))], axis=1)


# speedup vs baseline: 3.0937x; 1.4150x over previous
import functools

import jax
import jax.numpy as jnp
from jax import lax
from jax.experimental import pallas as pl
from jax.experimental.pallas import tpu as pltpu
from jax.experimental.pallas import tpu_sc as plsc

N_NODES = 10000
N_EDGES = 320000
D_FEAT = 128
MAX_HOP = 3

NC = 2
NS = 16
N_PAD = 10240
NODES_PER_CORE = N_PAD // NC
ROWS_PER_TILE = NODES_PER_CORE // NS
DUMP_ROW = NODES_PER_CORE
ACC_ROWS = NODES_PER_CORE + 8
EDGES_PER_TILE = N_EDGES // NS
CHUNK = 80
CHUNKS_PER_TILE = EDGES_PER_TILE // CHUNK
CHUNKS_PAD = CHUNKS_PER_TILE + 2


def _sc_body(table, src_idx, dst_idx, zeros, out, acc, rows, srci, dsti,
             sem0, sem1):
    c = lax.axis_index("c")
    s = lax.axis_index("s")
    sems = (sem0, sem1)

    pltpu.sync_copy(src_idx.at[c, s], srci)
    pltpu.sync_copy(dst_idx.at[s], dsti)

    row0 = s * ROWS_PER_TILE
    out_row0 = c * NODES_PER_CORE + s * ROWS_PER_TILE

    pltpu.sync_copy(zeros, acc.at[pl.ds(row0, ROWS_PER_TILE)])
    plsc.subcore_barrier()

    def drain(b):
        pltpu.make_async_copy(zeros.at[pl.ds(0, CHUNK)], rows.at[b],
                              sems[b]).wait()

    pltpu.async_copy(table.at[dsti.at[0]], rows.at[0], sems[0])

    def ring_body(i, _):
        for b in range(2):
            k = 2 * i + b
            pltpu.async_copy(table.at[dsti.at[k + 1]], rows.at[1 - b],
                             sems[1 - b])
            drain(b)
            pltpu.sync_copy(rows.at[b], acc.at[srci.at[k]], add=True)
        return ()

    lax.fori_loop(0, CHUNKS_PER_TILE // 2, ring_body, ())
    drain(0)

    plsc.subcore_barrier()
    pltpu.sync_copy(acc.at[pl.ds(row0, ROWS_PER_TILE)],
                    out.at[pl.ds(out_row0, ROWS_PER_TILE)])


@functools.partial(
    pl.kernel,
    out_type=jax.ShapeDtypeStruct((N_PAD, D_FEAT), jnp.float32),
    mesh=plsc.VectorSubcoreMesh(core_axis_name="c", subcore_axis_name="s"),
    scratch_types=[
        pltpu.VMEM_SHARED((ACC_ROWS, D_FEAT), jnp.float32),
        pltpu.VMEM((2, CHUNK, D_FEAT), jnp.float32),
        pltpu.VMEM((CHUNKS_PER_TILE, CHUNK), jnp.int32),
        pltpu.VMEM((CHUNKS_PAD, CHUNK), jnp.int32),
        pltpu.SemaphoreType.DMA,
        pltpu.SemaphoreType.DMA,
    ],
)
def _hop_kernel(table, src_idx, dst_idx, zeros, out, acc, rows, srci, dsti,
                sem0, sem1):
    _sc_body(table, src_idx, dst_idx, zeros, out, acc, rows, srci, dsti,
             sem0, sem1)


def kernel(x, edge_index, num_nodes):
    del num_nodes
    src = edge_index[0]
    dst = edge_index[1]
    core = src // NODES_PER_CORE
    local = src - core * NODES_PER_CORE
    srcm = jnp.stack(
        [jnp.where(core == c, local, DUMP_ROW) for c in range(NC)]
    ).reshape(NC, NS, CHUNKS_PER_TILE, CHUNK)
    dst3 = dst.reshape(NS, CHUNKS_PER_TILE, CHUNK)
    dst3 = jnp.pad(dst3, ((0, 0), (0, CHUNKS_PAD - CHUNKS_PER_TILE), (0, 0)))
    zeros = jnp.zeros((ROWS_PER_TILE, D_FEAT), jnp.float32)

    table = jnp.pad(x, ((0, N_PAD - N_NODES), (0, 0)))
    hops = []
    for _ in range(MAX_HOP):
        table = _hop_kernel(table, srcm, dst3, zeros)
        hops.append(table[:N_NODES])
    y = jnp.stack(hops)
    return jnp.concatenate([x[:, None], jnp.transpose(y, (1, 0, 2))], axis=1)


# speedup vs baseline: 3.1037x; 1.0032x over previous
import functools

import jax
import jax.numpy as jnp
from jax import lax
from jax.experimental import pallas as pl
from jax.experimental.pallas import tpu as pltpu
from jax.experimental.pallas import tpu_sc as plsc

N_NODES = 10000
N_EDGES = 320000
D_FEAT = 128
MAX_HOP = 3

NC = 2
NS = 16
N_PAD = 10240
NODES_PER_CORE = N_PAD // NC
ROWS_PER_TILE = NODES_PER_CORE // NS
DUMP_ROW = NODES_PER_CORE
ACC_ROWS = NODES_PER_CORE + 8
EDGES_PER_TILE = N_EDGES // NS
CHUNK = 80
CHUNKS_PER_TILE = EDGES_PER_TILE // CHUNK
CHUNKS_PAD = CHUNKS_PER_TILE + 2


def _sc_body(table, src_idx, dst_idx, zeros, out, acc, rows, srci, dsti,
             sem0, sem1):
    c = lax.axis_index("c")
    s = lax.axis_index("s")
    sems = (sem0, sem1)

    pltpu.sync_copy(src_idx.at[c, s], srci)
    pltpu.sync_copy(dst_idx.at[s], dsti)

    row0 = s * ROWS_PER_TILE
    out_row0 = c * NODES_PER_CORE + s * ROWS_PER_TILE

    pltpu.sync_copy(zeros, acc.at[pl.ds(row0, ROWS_PER_TILE)])
    plsc.subcore_barrier()

    def drain(b):
        pltpu.make_async_copy(zeros.at[pl.ds(0, CHUNK)], rows.at[b],
                              sems[b]).wait()

    pltpu.async_copy(table.at[dsti.at[0]], rows.at[0], sems[0])

    def ring_body(i, _):
        for b in range(2):
            k = 2 * i + b
            pltpu.async_copy(table.at[dsti.at[k + 1]], rows.at[1 - b],
                             sems[1 - b])
            drain(b)
            pltpu.sync_copy(rows.at[b], acc.at[srci.at[k]], add=True)
        return ()

    lax.fori_loop(0, CHUNKS_PER_TILE // 2, ring_body, ())
    drain(0)

    plsc.subcore_barrier()
    pltpu.sync_copy(acc.at[pl.ds(row0, ROWS_PER_TILE)],
                    out.at[pl.ds(out_row0, ROWS_PER_TILE)])


@functools.partial(
    pl.kernel,
    out_type=jax.ShapeDtypeStruct((N_PAD, D_FEAT), jnp.float32),
    mesh=plsc.VectorSubcoreMesh(core_axis_name="c", subcore_axis_name="s"),
    scratch_types=[
        pltpu.VMEM_SHARED((ACC_ROWS, D_FEAT), jnp.float32),
        pltpu.VMEM((2, CHUNK, D_FEAT), jnp.float32),
        pltpu.VMEM((CHUNKS_PER_TILE, CHUNK), jnp.int32),
        pltpu.VMEM((CHUNKS_PAD, CHUNK), jnp.int32),
        pltpu.SemaphoreType.DMA,
        pltpu.SemaphoreType.DMA,
    ],
)
def _hop_kernel(table, src_idx, dst_idx, zeros, out, acc, rows, srci, dsti,
                sem0, sem1):
    _sc_body(table, src_idx, dst_idx, zeros, out, acc, rows, srci, dsti,
             sem0, sem1)


@functools.partial(
    pl.kernel,
    out_type=jax.ShapeDtypeStruct((N_PAD, D_FEAT), jnp.float32),
    mesh=plsc.VectorSubcoreMesh(core_axis_name="c", subcore_axis_name="s"),
    scratch_types=[pltpu.VMEM((ROWS_PER_TILE, D_FEAT), jnp.float32)],
)
def _linearize(xp, out, buf):
    c = lax.axis_index("c")
    s = lax.axis_index("s")
    r0 = (s * NC + c) * ROWS_PER_TILE
    pltpu.sync_copy(xp.at[pl.ds(r0, ROWS_PER_TILE)], buf)
    pltpu.sync_copy(buf, out.at[pl.ds(r0, ROWS_PER_TILE)])


def kernel(x, edge_index, num_nodes):
    del num_nodes
    src = edge_index[0]
    dst = edge_index[1]
    core = src // NODES_PER_CORE
    local = src - core * NODES_PER_CORE
    srcm = jnp.stack(
        [jnp.where(core == c, local, DUMP_ROW) for c in range(NC)]
    ).reshape(NC, NS, CHUNKS_PER_TILE, CHUNK)
    dst3 = dst.reshape(NS, CHUNKS_PER_TILE, CHUNK)
    dst3 = jnp.pad(dst3, ((0, 0), (0, CHUNKS_PAD - CHUNKS_PER_TILE), (0, 0)))
    zeros = jnp.zeros((ROWS_PER_TILE, D_FEAT), jnp.float32)

    table = _linearize(jnp.pad(x, ((0, N_PAD - N_NODES), (0, 0))))
    hops = []
    for _ in range(MAX_HOP):
        table = _hop_kernel(table, srcm, dst3, zeros)
        hops.append(table[:N_NODES])
    y = jnp.stack(hops)
    return jnp.concatenate([x[:, None], jnp.transpose(y, (1, 0, 2))], axis=1)
